# baseline (speedup 1.0000x reference)
import functools

import jax
import jax.numpy as jnp
from jax import lax
from jax.experimental import pallas as pl
from jax.experimental.pallas import tpu as pltpu
from jax.experimental.pallas import tpu_sc as plsc

NC = 2
NS = 16
NW = NC * NS
B = 16384
D = 64
BPW = B // NW
CHUNK = 128
NCHUNK = BPW // CHUNK


def _sc_body(xu_hbm, xm_hbm, iu_hbm, im_hbm, out_hbm,
             iu_v, im_v, ur_v, mr_v, o_v, sem_u, sem_m):
    wid = lax.axis_index("s") * NC + lax.axis_index("c")
    base = wid * BPW

    pltpu.sync_copy(iu_hbm.at[wid], iu_v)
    pltpu.sync_copy(im_hbm.at[wid], im_v)

    copies = []
    for j in range(NCHUNK):
        copies.append(pltpu.async_copy(
            xu_hbm.at[iu_v.at[j]], ur_v.at[pl.ds(j * CHUNK, CHUNK)], sem_u))
        copies.append(pltpu.async_copy(
            xm_hbm.at[im_v.at[j]], mr_v.at[pl.ds(j * CHUNK, CHUNK)], sem_m))
    for cp in copies:
        cp.wait()

    def group_step(g, carry):
        e_ids = g * 16 + lax.iota(jnp.int32, 16)

        def dim_step(d, acc):
            d_ids = jnp.full((16,), d, jnp.int32)
            pu = plsc.load_gather(ur_v, [e_ids, d_ids])
            pm = plsc.load_gather(mr_v, [e_ids, d_ids])
            return acc + pu * pm

        acc = lax.fori_loop(0, D, dim_step, jnp.zeros((16,), jnp.float32))
        o_v[pl.ds(g * 16, 16)] = acc
        return carry

    lax.fori_loop(0, BPW // 16, group_step, 0)

    pltpu.sync_copy(o_v, out_hbm.at[pl.ds(base, BPW)])


@jax.jit
def _run(x_user, x_movie, iu, im):
    mesh = plsc.VectorSubcoreMesh(
        core_axis_name="c", subcore_axis_name="s",
        num_cores=NC, num_subcores=NS)
    f = pl.kernel(
        _sc_body,
        out_type=jax.ShapeDtypeStruct((B,), jnp.float32),
        mesh=mesh,
        scratch_types=[
            pltpu.VMEM((NCHUNK, CHUNK), jnp.int32),
            pltpu.VMEM((NCHUNK, CHUNK), jnp.int32),
            pltpu.VMEM((BPW, D), jnp.float32),
            pltpu.VMEM((BPW, D), jnp.float32),
            pltpu.VMEM((BPW,), jnp.float32),
            pltpu.SemaphoreType.DMA,
            pltpu.SemaphoreType.DMA,
        ],
        compiler_params=pltpu.CompilerParams(
            needs_layout_passes=False, use_tc_tiling_on_sc=False),
    )
    return f(x_user, x_movie, iu, im)


def kernel(x_user, x_movie, edge_label_index):
    idx = edge_label_index.astype(jnp.int32)
    iu = idx[0].reshape(NW, NCHUNK, CHUNK)
    im = idx[1].reshape(NW, NCHUNK, CHUNK)
    return _run(x_user, x_movie, iu, im)
